# baseline (speedup 1.0000x reference)
import functools

import jax
import jax.numpy as jnp
from jax import lax
from jax.experimental import pallas as pl
from jax.experimental.pallas import tpu as pltpu
from jax.experimental.pallas import tpu_sc as plsc

N = 10000
D = 256
E = 160000

NC = 2
NS = 16
DH = D // NC

CH = 128
NCHUNK = 80
HC = NCHUNK // 2
EPT = NCHUNK * CH
E_PAD = NS * EPT
PAD = E_PAD - E

N_ACC = 10240
RZ = N_ACC // NS


def _sc_aggregate(xflat, src5, dst4, zrows, zdeg, ones_row):
  mesh = plsc.VectorSubcoreMesh(
      core_axis_name="c", subcore_axis_name="s", num_cores=NC,
      num_subcores=NS)

  @functools.partial(
      pl.kernel,
      out_type=(
          jax.ShapeDtypeStruct((NC, N_ACC, DH), jnp.float32),
          jax.ShapeDtypeStruct((NC, N_ACC), jnp.float32),
      ),
      mesh=mesh,
      scratch_types=[
          pltpu.VMEM((HC, CH), jnp.int32),
          pltpu.VMEM((HC, CH), jnp.int32),
          pltpu.VMEM((2, CH, DH), jnp.float32),
          pltpu.VMEM((CH,), jnp.float32),
          pltpu.VMEM_SHARED((N_ACC, DH), jnp.float32),
          pltpu.VMEM_SHARED((N_ACC,), jnp.float32),
          pltpu.SemaphoreType.DMA((2,)),
          pltpu.SemaphoreType.DMA((2,)),
      ],
  )
  def k(xflat_h, src_h, dst_h, zr_h, zd_h, on_h, agg_h, deg_h,
        srcv, dstv, rows, onesv, acc, degs, gsem, ssem):
    c = lax.axis_index("c")
    s = lax.axis_index("s")

    pltpu.sync_copy(zr_h, acc.at[pl.ds(s * RZ, RZ)])
    pltpu.sync_copy(zd_h, degs.at[pl.ds(s * RZ, RZ)])
    pltpu.sync_copy(on_h, onesv)

    plsc.subcore_barrier()

    def gather(g, p):
      pltpu.async_copy(xflat_h.at[srcv.at[g]], rows.at[p], gsem.at[p])

    def wait_gather(g, p):
      pltpu.make_async_copy(xflat_h.at[srcv.at[g]], rows.at[p],
                            gsem.at[p]).wait()

    def scatter(g, p):
      pltpu.async_copy(rows.at[p], acc.at[dstv.at[g]], ssem.at[p], add=True)

    def wait_scatter(g, p):
      pltpu.make_async_copy(rows.at[p], acc.at[dstv.at[g]], ssem.at[p]).wait()

    for h in range(2):
      pltpu.sync_copy(src_h.at[c, s, h], srcv)
      pltpu.sync_copy(dst_h.at[s, h], dstv)

      gather(0, 0)

      def body(g, carry):
        p = g % 2
        wait_gather(g, p)

        @pl.when(g >= 1)
        def _():
          wait_scatter(g - 1, 1 - p)

        @pl.when(g + 1 < HC)
        def _():
          gather(g + 1, 1 - p)

        scatter(g, p)

        @pl.when(c == h)
        def _():
          pltpu.sync_copy(onesv, degs.at[dstv.at[g]], add=True)

        return carry

      lax.fori_loop(0, HC, body, 0)
      wait_scatter(HC - 1, (HC - 1) % 2)

    plsc.subcore_barrier()

    pltpu.sync_copy(acc.at[pl.ds(s * RZ, RZ)],
                    agg_h.at[c, pl.ds(s * RZ, RZ)])
    pltpu.sync_copy(degs.at[pl.ds(s * RZ, RZ)],
                    deg_h.at[c, pl.ds(s * RZ, RZ)])

  return k(xflat, src5, dst4, zrows, zdeg, ones_row)


def _tc_self_body(x_ref, ws_ref, b_ref, o_ref):
  o_ref[...] = jnp.dot(x_ref[...], ws_ref[...],
                       preferred_element_type=jnp.float32) + b_ref[...]


def _tc_self(x, ws, b2):
  R = 1000
  return pl.pallas_call(
      _tc_self_body,
      grid=(N // R,),
      in_specs=[
          pl.BlockSpec((R, D), lambda i: (i, 0)),
          pl.BlockSpec((D, D), lambda i: (0, 0)),
          pl.BlockSpec((1, D), lambda i: (0, 0)),
      ],
      out_specs=pl.BlockSpec((R, D), lambda i: (i, 0)),
      out_shape=jax.ShapeDtypeStruct((N, D), jnp.float32),
  )(x, ws, b2)


def _tc_body(hs_ref, a_ref, deg_ref, wn_ref, o_ref):
  hn = jnp.dot(a_ref[0], wn_ref[0], preferred_element_type=jnp.float32)
  hn = hn + jnp.dot(a_ref[1], wn_ref[1], preferred_element_type=jnp.float32)
  deg = deg_ref[0] + deg_ref[1]
  scale = 1.0 / jnp.maximum(deg, 1.0)
  o_ref[...] = jnp.maximum(hs_ref[...] + hn * scale, 0.0)


def _tc_combine(hself, agg, deg3, wn2):
  R = 1000
  return pl.pallas_call(
      _tc_body,
      grid=(N // R,),
      in_specs=[
          pl.BlockSpec((R, D), lambda i: (i, 0)),
          pl.BlockSpec((NC, R, DH), lambda i: (0, i, 0)),
          pl.BlockSpec((NC, R, 1), lambda i: (0, i, 0)),
          pl.BlockSpec((NC, DH, D), lambda i: (0, 0, 0)),
      ],
      out_specs=pl.BlockSpec((R, D), lambda i: (i, 0)),
      out_shape=jax.ShapeDtypeStruct((N, D), jnp.float32),
  )(hself, agg, deg3, wn2)


def kernel(x, edge_index, W_self, W_neigh, b):
  src = edge_index[0].astype(jnp.int32)
  dst = edge_index[1].astype(jnp.int32)
  src_p = jnp.concatenate([src, jnp.zeros((PAD,), jnp.int32)])
  dst_p = jnp.concatenate([dst, jnp.full((PAD,), N, jnp.int32)])
  src5 = jnp.stack([2 * src_p, 2 * src_p + 1]).reshape(NC, NS, 2, HC, CH)
  dst4 = dst_p.reshape(NS, 2, HC, CH)
  xflat = x.reshape(NC * N, DH)

  zrows = jnp.zeros((RZ, DH), jnp.float32)
  zdeg = jnp.zeros((RZ,), jnp.float32)
  ones_row = jnp.ones((CH,), jnp.float32)

  agg, degp = _sc_aggregate(xflat, src5, dst4, zrows, zdeg, ones_row)
  hself = _tc_self(x, W_self, b.reshape(1, D))

  deg3 = degp.reshape(NC, N_ACC, 1)
  wn2 = W_neigh.reshape(NC, DH, D)
  return _tc_combine(hself, agg, deg3, wn2)


# speedup vs baseline: 1.0695x; 1.0695x over previous
import functools

import jax
import jax.numpy as jnp
from jax import lax
from jax.experimental import pallas as pl
from jax.experimental.pallas import tpu as pltpu
from jax.experimental.pallas import tpu_sc as plsc

N = 10000
D = 256
E = 160000

NC = 2
NS = 16
DH = D // NC

CH = 128
NCHUNK = 80
HC = NCHUNK // 2
EPT = NCHUNK * CH
E_PAD = NS * EPT
PAD = E_PAD - E

N_ACC = 10240
RZ = N_ACC // NS


def _sc_aggregate(xflat, src5, dst4, zrows, zdeg, ones_row):
  mesh = plsc.VectorSubcoreMesh(
      core_axis_name="c", subcore_axis_name="s", num_cores=NC,
      num_subcores=NS)

  @functools.partial(
      pl.kernel,
      out_type=(
          jax.ShapeDtypeStruct((NC, N_ACC, DH), jnp.float32),
          jax.ShapeDtypeStruct((NC, N_ACC), jnp.float32),
      ),
      mesh=mesh,
      scratch_types=[
          pltpu.VMEM((HC, CH), jnp.int32),
          pltpu.VMEM((HC, CH), jnp.int32),
          pltpu.VMEM((2, CH, DH), jnp.float32),
          pltpu.VMEM((CH,), jnp.float32),
          pltpu.VMEM_SHARED((N_ACC, DH), jnp.float32),
          pltpu.VMEM_SHARED((N_ACC,), jnp.float32),
          pltpu.SemaphoreType.DMA((2,)),
          pltpu.SemaphoreType.DMA((2,)),
      ],
  )
  def k(xflat_h, src_h, dst_h, zr_h, zd_h, on_h, agg_h, deg_h,
        srcv, dstv, rows, onesv, acc, degs, gsem, ssem):
    c = lax.axis_index("c")
    s = lax.axis_index("s")

    pltpu.sync_copy(zr_h, acc.at[pl.ds(s * RZ, RZ)])
    pltpu.sync_copy(zd_h, degs.at[pl.ds(s * RZ, RZ)])
    pltpu.sync_copy(on_h, onesv)

    plsc.subcore_barrier()

    def gather(g, p):
      pltpu.async_copy(xflat_h.at[srcv.at[g]], rows.at[p], gsem.at[p])

    def wait_gather(g, p):
      pltpu.make_async_copy(xflat_h.at[srcv.at[g]], rows.at[p],
                            gsem.at[p]).wait()

    def scatter(g, p):
      pltpu.async_copy(rows.at[p], acc.at[dstv.at[g]], ssem.at[p], add=True)

    def wait_scatter(g, p):
      pltpu.make_async_copy(rows.at[p], acc.at[dstv.at[g]], ssem.at[p]).wait()

    for h in range(2):
      pltpu.sync_copy(src_h.at[c, s, h], srcv)
      pltpu.sync_copy(dst_h.at[s, h], dstv)

      gather(0, 0)

      def body(g, carry):
        p = g % 2

        @pl.when(g >= 1)
        def _():
          wait_scatter(g - 1, 1 - p)

        @pl.when(g + 1 < HC)
        def _():
          gather(g + 1, 1 - p)

        wait_gather(g, p)
        scatter(g, p)

        @pl.when(c == h)
        def _():
          pltpu.sync_copy(onesv, degs.at[dstv.at[g]], add=True)

        return carry

      lax.fori_loop(0, HC, body, 0)
      wait_scatter(HC - 1, (HC - 1) % 2)

    plsc.subcore_barrier()

    pltpu.sync_copy(acc.at[pl.ds(s * RZ, RZ)],
                    agg_h.at[c, pl.ds(s * RZ, RZ)])
    pltpu.sync_copy(degs.at[pl.ds(s * RZ, RZ)],
                    deg_h.at[c, pl.ds(s * RZ, RZ)])

  return k(xflat, src5, dst4, zrows, zdeg, ones_row)


def _tc_body(x_ref, a_ref, deg_ref, ws_ref, wn_ref, b_ref, o_ref):
  h = jnp.dot(x_ref[...], ws_ref[...], preferred_element_type=jnp.float32)
  hn = jnp.dot(a_ref[0], wn_ref[0], preferred_element_type=jnp.float32)
  hn = hn + jnp.dot(a_ref[1], wn_ref[1], preferred_element_type=jnp.float32)
  deg = deg_ref[0] + deg_ref[1]
  scale = 1.0 / jnp.maximum(deg, 1.0)
  o_ref[...] = jnp.maximum(h + hn * scale + b_ref[...], 0.0)


def _tc_combine(x, agg, deg3, ws, wn2, b2):
  R = 1000
  return pl.pallas_call(
      _tc_body,
      grid=(N // R,),
      in_specs=[
          pl.BlockSpec((R, D), lambda i: (i, 0)),
          pl.BlockSpec((NC, R, DH), lambda i: (0, i, 0)),
          pl.BlockSpec((NC, R, 1), lambda i: (0, i, 0)),
          pl.BlockSpec((D, D), lambda i: (0, 0)),
          pl.BlockSpec((NC, DH, D), lambda i: (0, 0, 0)),
          pl.BlockSpec((1, D), lambda i: (0, 0)),
      ],
      out_specs=pl.BlockSpec((R, D), lambda i: (i, 0)),
      out_shape=jax.ShapeDtypeStruct((N, D), jnp.float32),
  )(x, agg, deg3, ws, wn2, b2)


def kernel(x, edge_index, W_self, W_neigh, b):
  src = edge_index[0].astype(jnp.int32)
  dst = edge_index[1].astype(jnp.int32)
  src_p = jnp.concatenate([src, jnp.zeros((PAD,), jnp.int32)])
  dst_p = jnp.concatenate([dst, jnp.full((PAD,), N, jnp.int32)])
  src5 = jnp.stack([2 * src_p, 2 * src_p + 1]).reshape(NC, NS, 2, HC, CH)
  dst4 = dst_p.reshape(NS, 2, HC, CH)
  xflat = x.reshape(NC * N, DH)

  zrows = jnp.zeros((RZ, DH), jnp.float32)
  zdeg = jnp.zeros((RZ,), jnp.float32)
  ones_row = jnp.ones((CH,), jnp.float32)

  agg, degp = _sc_aggregate(xflat, src5, dst4, zrows, zdeg, ones_row)

  deg3 = degp.reshape(NC, N_ACC, 1)
  wn2 = W_neigh.reshape(NC, DH, D)
  b2 = b.reshape(1, D)
  return _tc_combine(x, agg, deg3, W_self, wn2, b2)


# speedup vs baseline: 1.0697x; 1.0002x over previous
import functools

import jax
import jax.numpy as jnp
from jax import lax
from jax.experimental import pallas as pl
from jax.experimental.pallas import tpu as pltpu
from jax.experimental.pallas import tpu_sc as plsc

N = 10000
D = 256
E = 160000

NC = 2
NS = 16
DH = D // NC

CH = 128
NCHUNK = 80
HC = NCHUNK // 2
EPT = NCHUNK * CH
E_PAD = NS * EPT
PAD = E_PAD - E

N_ACC = 10240
RZ = N_ACC // NS


def _sc_aggregate(xflat, src5, dst4, zrows, zdeg, ones_row):
  mesh = plsc.VectorSubcoreMesh(
      core_axis_name="c", subcore_axis_name="s", num_cores=NC,
      num_subcores=NS)

  @functools.partial(
      pl.kernel,
      out_type=(
          jax.ShapeDtypeStruct((NC, N_ACC, DH), jnp.float32),
          jax.ShapeDtypeStruct((NC, N_ACC), jnp.float32),
      ),
      mesh=mesh,
      scratch_types=[
          pltpu.VMEM((HC, CH), jnp.int32),
          pltpu.VMEM((HC, CH), jnp.int32),
          pltpu.VMEM((2, CH, DH), jnp.float32),
          pltpu.VMEM((CH,), jnp.float32),
          pltpu.VMEM_SHARED((N_ACC, DH), jnp.float32),
          pltpu.VMEM_SHARED((N_ACC,), jnp.float32),
          pltpu.SemaphoreType.DMA((2,)),
          pltpu.SemaphoreType.DMA((2,)),
          pltpu.SemaphoreType.DMA,
      ],
  )
  def k(xflat_h, src_h, dst_h, zr_h, zd_h, on_h, agg_h, deg_h,
        srcv, dstv, rows, onesv, acc, degs, gsem, ssem, dsem):
    c = lax.axis_index("c")
    s = lax.axis_index("s")

    pltpu.sync_copy(zr_h, acc.at[pl.ds(s * RZ, RZ)])
    pltpu.sync_copy(zd_h, degs.at[pl.ds(s * RZ, RZ)])
    pltpu.sync_copy(on_h, onesv)

    plsc.subcore_barrier()

    def gather(g, p):
      pltpu.async_copy(xflat_h.at[srcv.at[g]], rows.at[p], gsem.at[p])

    def wait_gather(g, p):
      pltpu.make_async_copy(xflat_h.at[srcv.at[g]], rows.at[p],
                            gsem.at[p]).wait()

    def scatter(g, p):
      pltpu.async_copy(rows.at[p], acc.at[dstv.at[g]], ssem.at[p], add=True)

    def wait_scatter(g, p):
      pltpu.make_async_copy(rows.at[p], acc.at[dstv.at[g]], ssem.at[p]).wait()

    for h in range(2):
      pltpu.sync_copy(src_h.at[c, s, h], srcv)
      pltpu.sync_copy(dst_h.at[s, h], dstv)

      gather(0, 0)

      def body(g, carry):
        p = g % 2

        @pl.when(g >= 1)
        def _():
          wait_scatter(g - 1, 1 - p)

        @pl.when(g + 1 < HC)
        def _():
          gather(g + 1, 1 - p)

        wait_gather(g, p)
        scatter(g, p)

        @pl.when(c == h)
        def _():
          pltpu.async_copy(onesv, degs.at[dstv.at[g]], dsem, add=True)

        return carry

      lax.fori_loop(0, HC, body, 0)
      wait_scatter(HC - 1, (HC - 1) % 2)

      @pl.when(c == h)
      def _():
        def drain(g, carry):
          pltpu.make_async_copy(onesv, degs.at[dstv.at[g]], dsem).wait()
          return carry

        lax.fori_loop(0, HC, drain, 0)

    plsc.subcore_barrier()

    pltpu.sync_copy(acc.at[pl.ds(s * RZ, RZ)],
                    agg_h.at[c, pl.ds(s * RZ, RZ)])
    pltpu.sync_copy(degs.at[pl.ds(s * RZ, RZ)],
                    deg_h.at[c, pl.ds(s * RZ, RZ)])

  return k(xflat, src5, dst4, zrows, zdeg, ones_row)


def _tc_body(x_ref, a_ref, deg_ref, ws_ref, wn_ref, b_ref, o_ref):
  h = jnp.dot(x_ref[...], ws_ref[...], preferred_element_type=jnp.float32)
  hn = jnp.dot(a_ref[0], wn_ref[0], preferred_element_type=jnp.float32)
  hn = hn + jnp.dot(a_ref[1], wn_ref[1], preferred_element_type=jnp.float32)
  deg = deg_ref[0] + deg_ref[1]
  scale = 1.0 / jnp.maximum(deg, 1.0)
  o_ref[...] = jnp.maximum(h + hn * scale + b_ref[...], 0.0)


def _tc_combine(x, agg, deg3, ws, wn2, b2):
  R = 1000
  return pl.pallas_call(
      _tc_body,
      grid=(N // R,),
      in_specs=[
          pl.BlockSpec((R, D), lambda i: (i, 0)),
          pl.BlockSpec((NC, R, DH), lambda i: (0, i, 0)),
          pl.BlockSpec((NC, R, 1), lambda i: (0, i, 0)),
          pl.BlockSpec((D, D), lambda i: (0, 0)),
          pl.BlockSpec((NC, DH, D), lambda i: (0, 0, 0)),
          pl.BlockSpec((1, D), lambda i: (0, 0)),
      ],
      out_specs=pl.BlockSpec((R, D), lambda i: (i, 0)),
      out_shape=jax.ShapeDtypeStruct((N, D), jnp.float32),
  )(x, agg, deg3, ws, wn2, b2)


def kernel(x, edge_index, W_self, W_neigh, b):
  src = edge_index[0].astype(jnp.int32)
  dst = edge_index[1].astype(jnp.int32)
  src_p = jnp.concatenate([src, jnp.zeros((PAD,), jnp.int32)])
  dst_p = jnp.concatenate([dst, jnp.full((PAD,), N, jnp.int32)])
  src5 = jnp.stack([2 * src_p, 2 * src_p + 1]).reshape(NC, NS, 2, HC, CH)
  dst4 = dst_p.reshape(NS, 2, HC, CH)
  xflat = x.reshape(NC * N, DH)

  zrows = jnp.zeros((RZ, DH), jnp.float32)
  zdeg = jnp.zeros((RZ,), jnp.float32)
  ones_row = jnp.ones((CH,), jnp.float32)

  agg, degp = _sc_aggregate(xflat, src5, dst4, zrows, zdeg, ones_row)

  deg3 = degp.reshape(NC, N_ACC, 1)
  wn2 = W_neigh.reshape(NC, DH, D)
  b2 = b.reshape(1, D)
  return _tc_combine(x, agg, deg3, W_self, wn2, b2)
